# baseline (speedup 1.0000x reference)
import functools
import jax
import jax.numpy as jnp
from jax import lax
from jax.experimental import pallas as pl
from jax.experimental.pallas import tpu as pltpu
from jax.experimental.pallas import tpu_sc as plsc

NS = 64
NI = 128
NRAYS = 65536
NWORK = 32
CHUNK = 128
UNROLL = 2
RPW = NRAYS // NWORK


def _tvals(x):
    return 189.0 / (94.5 - x)


def _sc_body(w_hbm, out_hbm, wbuf, obuf, cdfbuf, histbuf, tvbuf):
    wid = lax.axis_index("s") * 2 + lax.axis_index("c")
    base = wid * RPW

    iota = lax.iota(jnp.int32, 16)
    iotaf = iota.astype(jnp.float32)
    zeros = jnp.zeros((16,), jnp.float32)
    ones = jnp.ones((16,), jnp.float32)

    for k in range(4):
        tvbuf[pl.ds(k * 16, 16)] = _tvals(iotaf + (16.0 * k))

    def chunk_body(ci, _):
        cbase = base + ci * CHUNK
        pltpu.sync_copy(w_hbm.at[pl.ds(cbase, CHUNK)], wbuf)

        @plsc.parallel_loop(0, CHUNK, 1, unroll=UNROLL)
        def ray_body(r):
            r64 = r * 64
            r144 = r * 144
            r64v = jnp.full((16,), r64, jnp.int32)
            r144v = jnp.full((16,), r144, jnp.int32)
            pv = []
            for k in range(4):
                v = wbuf[r, pl.ds(k * 16, 16)] + 1e-5
                if k == 0:
                    v = jnp.where(iota == 0, 0.0, v)
                if k == 3:
                    v = jnp.where(iota == 15, 0.0, v)
                pv.append(v)
            v01 = pv[0] + pv[1]
            v012 = v01 + pv[2]
            v0123 = v012 + pv[3]
            cars = [jnp.float32(0.0), jnp.sum(pv[0]), jnp.sum(v01),
                    jnp.sum(v012)]
            psum = jnp.sum(v0123)
            psumv = jnp.full((16,), psum)
            scale = 127.0 / psumv
            svregs = []
            for k in range(4):
                c = plsc.cumsum(pv[k]) + cars[k]
                cdfbuf[pl.ds(r64 + k * 16, 16)] = c
                y = c * scale
                s = y.astype(jnp.int32) + 1
                if k == 0:
                    s = jnp.where(iota == 0, 0, s)
                if k == 3:
                    s = jnp.where(iota == 15, 143, s)
                svregs.append(s)

            for k in range(8):
                histbuf[pl.ds(r144 + k * 16, 16)] = zeros
            for k in range(4):
                plsc.addupdate_scatter(histbuf, [svregs[k] + r144v], ones)

            hv = [histbuf[pl.ds(r144 + k * 16, 16)] for k in range(8)]
            acc = hv[0]
            hcars = [jnp.float32(0.0)]
            for k in range(1, 8):
                hcars.append(jnp.sum(acc))
                if k < 7:
                    acc = acc + hv[k]
            for k in range(8):
                inds = plsc.cumsum(hv[k]) + hcars[k]
                below = inds - 1.0
                above = jnp.minimum(inds, 62.0)
                bi = below.astype(jnp.int32)
                ai = above.astype(jnp.int32)
                c0 = plsc.load_gather(cdfbuf, [bi + r64v])
                c1 = plsc.load_gather(cdfbuf, [ai + r64v])
                b0 = plsc.load_gather(tvbuf, [bi])
                b1 = plsc.load_gather(tvbuf, [ai])
                uj = (iotaf + (16.0 * k)) * (1.0 / 127.0) * psumv
                gap = c1 - c0
                den = jnp.where(gap < 1e-5 * psumv, psumv, gap)
                tt = (uj - c0) / den
                obuf[r, pl.ds(k * 16, 16)] = b0 + tt * (b1 - b0)

        pltpu.sync_copy(obuf, out_hbm.at[pl.ds(cbase, CHUNK)])
        return _

    lax.fori_loop(0, RPW // CHUNK, chunk_body, 0)


def _sc_samples(weights):
    mesh = plsc.VectorSubcoreMesh(core_axis_name="c", subcore_axis_name="s")
    return pl.kernel(
        _sc_body,
        mesh=mesh,
        compiler_params=pltpu.CompilerParams(needs_layout_passes=False),
        out_type=jax.ShapeDtypeStruct((NRAYS, NI), jnp.float32),
        scratch_types=[
            pltpu.VMEM((CHUNK, NS), jnp.float32),
            pltpu.VMEM((CHUNK, NI), jnp.float32),
            pltpu.VMEM((CHUNK * NS,), jnp.float32),
            pltpu.VMEM((CHUNK * 144,), jnp.float32),
            pltpu.VMEM((NS,), jnp.float32),
        ],
    )(weights)


BTC = 256


def _vol_kernel(d_ref, c_ref, out_ref):
    dens = d_ref[...]
    si = lax.broadcasted_iota(jnp.int32, (1, NS), 1).astype(jnp.float32)
    tv = _tvals(si)
    dists = jnp.where(si >= NS - 1.0, 1e10, _tvals(si + 1.0) - tv)
    alpha = 1.0 - jnp.exp(-jnp.maximum(dens, 0.0) * dists)
    om = 1.0 - alpha + 1e-10
    cp = om
    s = 1
    while s < NS:
        pad = jnp.ones((BTC, s), jnp.float32)
        cp = cp * jnp.concatenate([pad, cp[:, : NS - s]], axis=1)
        s *= 2
    trans = jnp.concatenate([jnp.ones((BTC, 1), jnp.float32), cp[:, :63]],
                            axis=1)
    wv = alpha * trans
    cr = jnp.sum(wv * c_ref[0], axis=1, keepdims=True)
    cg = jnp.sum(wv * c_ref[1], axis=1, keepdims=True)
    cb = jnp.sum(wv * c_ref[2], axis=1, keepdims=True)
    depth = jnp.sum(wv * tv, axis=1, keepdims=True)
    acc = jnp.sum(wv, axis=1, keepdims=True)

    out_ref[:, 0:1] = cr
    out_ref[:, 1:2] = cg
    out_ref[:, 2:3] = cb
    out_ref[:, 3:4] = depth
    out_ref[:, 4:5] = acc
    out_ref[:, 5:69] = wv


def _tc_volumetric(densities, colors):
    n = densities.shape[0]
    dens = densities[..., 0]
    c3 = jnp.transpose(colors, (2, 0, 1))
    return pl.pallas_call(
        _vol_kernel,
        grid=(n // BTC,),
        in_specs=[
            pl.BlockSpec((BTC, NS), lambda i: (i, 0)),
            pl.BlockSpec((3, BTC, NS), lambda i: (0, i, 0)),
        ],
        out_specs=pl.BlockSpec((BTC, 69), lambda i: (i, 0)),
        out_shape=jax.ShapeDtypeStruct((n, 69), jnp.float32),
        compiler_params=pltpu.CompilerParams(
            dimension_semantics=("parallel",)),
    )(dens, c3)


def _asm_kernel(s_ref, v_ref, out_ref):
    out_ref[:, 0:NI] = s_ref[...]
    out_ref[:, NI:197] = v_ref[...]


def _tc_assemble(samples, vol):
    n = samples.shape[0]
    return pl.pallas_call(
        _asm_kernel,
        grid=(n // BTC,),
        in_specs=[
            pl.BlockSpec((BTC, NI), lambda i: (i, 0)),
            pl.BlockSpec((BTC, 69), lambda i: (i, 0)),
        ],
        out_specs=pl.BlockSpec((BTC, 197), lambda i: (i, 0)),
        out_shape=jax.ShapeDtypeStruct((n, 197), jnp.float32),
        compiler_params=pltpu.CompilerParams(
            dimension_semantics=("parallel",)),
    )(samples, vol)


def kernel(origins, directions, weights, densities, colors):
    samples = _sc_samples(weights)
    vol = _tc_volumetric(densities, colors)
    return jnp.concatenate([samples, vol], axis=1)


# speedup vs baseline: 1.0988x; 1.0988x over previous
import functools
import jax
import jax.numpy as jnp
from jax import lax
from jax.experimental import pallas as pl
from jax.experimental.pallas import tpu as pltpu
from jax.experimental.pallas import tpu_sc as plsc

NS = 64
NI = 128
NRAYS = 65536
NWORK = 32
CHUNK = 128
UNROLL = 2
RPW = NRAYS // NWORK


def _tvals(x):
    return 189.0 / (94.5 - x)


def _sc_body(w_hbm, out_hbm, wbuf, obuf, cdfbuf, histbuf, tvbuf):
    wid = lax.axis_index("s") * 2 + lax.axis_index("c")
    base = wid * RPW

    iota = lax.iota(jnp.int32, 16)
    iotaf = iota.astype(jnp.float32)
    zeros = jnp.zeros((16,), jnp.float32)
    ones = jnp.ones((16,), jnp.float32)

    for k in range(4):
        tvbuf[pl.ds(k * 16, 16)] = _tvals(iotaf + (16.0 * k))

    def chunk_body(ci, _):
        cbase = base + ci * CHUNK
        pltpu.sync_copy(w_hbm.at[pl.ds(cbase, CHUNK)], wbuf)

        @plsc.parallel_loop(0, CHUNK, 1, unroll=UNROLL)
        def ray_body(r):
            r64 = r * 64
            r144 = r * 144
            r64v = jnp.full((16,), r64, jnp.int32)
            r144v = jnp.full((16,), r144, jnp.int32)
            pv = []
            for k in range(4):
                v = wbuf[r, pl.ds(k * 16, 16)] + 1e-5
                if k == 0:
                    v = jnp.where(iota == 0, 0.0, v)
                if k == 3:
                    v = jnp.where(iota == 15, 0.0, v)
                pv.append(v)
            v01 = pv[0] + pv[1]
            v012 = v01 + pv[2]
            v0123 = v012 + pv[3]
            cars = [jnp.float32(0.0), jnp.sum(pv[0]), jnp.sum(v01),
                    jnp.sum(v012)]
            psum = jnp.sum(v0123)
            psumv = jnp.full((16,), psum)
            scale = 127.0 / psumv
            svregs = []
            for k in range(4):
                c = plsc.cumsum(pv[k]) + cars[k]
                cdfbuf[pl.ds(r64 + k * 16, 16)] = c
                y = c * scale
                yi = y.astype(jnp.int32)
                s = jnp.where(yi.astype(jnp.float32) < y, yi + 1, yi)
                if k == 3:
                    s = jnp.where(iota == 15, 143, s)
                svregs.append(s)

            for k in range(9):
                histbuf[pl.ds(r144 + k * 16, 16)] = zeros
            for k in range(4):
                plsc.addupdate_scatter(histbuf, [svregs[k] + r144v], ones)

            hv = [histbuf[pl.ds(r144 + k * 16, 16)] for k in range(8)]
            acc = hv[0]
            hcars = [jnp.float32(0.0)]
            for k in range(1, 8):
                hcars.append(jnp.sum(acc))
                if k < 7:
                    acc = acc + hv[k]
            for k in range(8):
                inds = plsc.cumsum(hv[k]) + hcars[k]
                below = inds - 1.0
                above = jnp.minimum(inds, 62.0)
                bi = below.astype(jnp.int32)
                ai = above.astype(jnp.int32)
                c0 = plsc.load_gather(cdfbuf, [bi + r64v])
                c1 = plsc.load_gather(cdfbuf, [ai + r64v])
                b0 = plsc.load_gather(tvbuf, [bi])
                b1 = plsc.load_gather(tvbuf, [ai])
                uj = (iotaf + (16.0 * k)) * (1.0 / 127.0) * psumv
                gap = c1 - c0
                den = jnp.where(gap < 1e-5 * psumv, psumv, gap)
                tt = (uj - c0) / den
                obuf[r, pl.ds(k * 16, 16)] = b0 + tt * (b1 - b0)

        pltpu.sync_copy(obuf, out_hbm.at[pl.ds(cbase, CHUNK)])
        return _

    lax.fori_loop(0, RPW // CHUNK, chunk_body, 0)


def _sc_samples(weights):
    mesh = plsc.VectorSubcoreMesh(core_axis_name="c", subcore_axis_name="s")
    return pl.kernel(
        _sc_body,
        mesh=mesh,
        compiler_params=pltpu.CompilerParams(needs_layout_passes=False),
        out_type=jax.ShapeDtypeStruct((NRAYS, NI), jnp.float32),
        scratch_types=[
            pltpu.VMEM((CHUNK, NS), jnp.float32),
            pltpu.VMEM((CHUNK, NI), jnp.float32),
            pltpu.VMEM((CHUNK * NS,), jnp.float32),
            pltpu.VMEM((CHUNK * 144,), jnp.float32),
            pltpu.VMEM((NS,), jnp.float32),
        ],
    )(weights)


BTC = 512


def _vol_kernel(d_ref, c_ref, out_ref):
    dens = d_ref[...]
    si = lax.broadcasted_iota(jnp.int32, (1, NS), 1).astype(jnp.float32)
    tv = _tvals(si)
    dists = jnp.where(si >= NS - 1.0, 1e10, _tvals(si + 1.0) - tv)
    alpha = 1.0 - jnp.exp(-jnp.maximum(dens, 0.0) * dists)
    om = 1.0 - alpha + 1e-10
    cp = om
    s = 1
    while s < NS:
        pad = jnp.ones((BTC, s), jnp.float32)
        cp = cp * jnp.concatenate([pad, cp[:, : NS - s]], axis=1)
        s *= 2
    trans = jnp.concatenate([jnp.ones((BTC, 1), jnp.float32), cp[:, :63]],
                            axis=1)
    wv = alpha * trans
    cr = jnp.sum(wv * c_ref[0], axis=1, keepdims=True)
    cg = jnp.sum(wv * c_ref[1], axis=1, keepdims=True)
    cb = jnp.sum(wv * c_ref[2], axis=1, keepdims=True)
    depth = jnp.sum(wv * tv, axis=1, keepdims=True)
    acc = jnp.sum(wv, axis=1, keepdims=True)

    out_ref[:, 0:1] = cr
    out_ref[:, 1:2] = cg
    out_ref[:, 2:3] = cb
    out_ref[:, 3:4] = depth
    out_ref[:, 4:5] = acc
    out_ref[:, 5:69] = wv


def _tc_volumetric(densities, colors):
    n = densities.shape[0]
    dens = densities[..., 0]
    c3 = jnp.transpose(colors, (2, 0, 1))
    return pl.pallas_call(
        _vol_kernel,
        grid=(n // BTC,),
        in_specs=[
            pl.BlockSpec((BTC, NS), lambda i: (i, 0)),
            pl.BlockSpec((3, BTC, NS), lambda i: (0, i, 0)),
        ],
        out_specs=pl.BlockSpec((BTC, 69), lambda i: (i, 0)),
        out_shape=jax.ShapeDtypeStruct((n, 69), jnp.float32),
        compiler_params=pltpu.CompilerParams(
            dimension_semantics=("parallel",)),
    )(dens, c3)


def _asm_kernel(s_ref, v_ref, out_ref):
    out_ref[:, 0:NI] = s_ref[...]
    out_ref[:, NI:197] = v_ref[...]


def _tc_assemble(samples, vol):
    n = samples.shape[0]
    return pl.pallas_call(
        _asm_kernel,
        grid=(n // BTC,),
        in_specs=[
            pl.BlockSpec((BTC, NI), lambda i: (i, 0)),
            pl.BlockSpec((BTC, 69), lambda i: (i, 0)),
        ],
        out_specs=pl.BlockSpec((BTC, 197), lambda i: (i, 0)),
        out_shape=jax.ShapeDtypeStruct((n, 197), jnp.float32),
        compiler_params=pltpu.CompilerParams(
            dimension_semantics=("parallel",)),
    )(samples, vol)


def kernel(origins, directions, weights, densities, colors):
    samples = _sc_samples(weights)
    vol = _tc_volumetric(densities, colors)
    return jnp.concatenate([samples, vol], axis=1)
